# Initial kernel scaffold; baseline (speedup 1.0000x reference)
import functools

import jax
import jax.numpy as jnp
import numpy as np
from jax import lax
from jax.experimental import pallas as pl
from jax.experimental.pallas import tpu as pltpu
from jax.experimental.pallas import tpu_sc as plsc

N_NODES = 10000
F = 128
O = 128
NUM_NEIGH = 10

NC = 2
NS = 16
NW = NC * NS

SUB = 8
IDX_PER_SUB = SUB * NUM_NEIGH
ROWS_PER_W = 320

_SEG = np.repeat(np.arange(SUB, dtype=np.int32), NUM_NEIGH)


def _tc_matmul(X, Wc, bc):
    blk = 2000

    def body(x_ref, w_ref, b_ref, z_ref, y_ref):
        zy = jnp.dot(x_ref[...], w_ref[...],
                     preferred_element_type=jnp.float32) + b_ref[...]
        z_ref[...] = zy[:, :O]
        y_ref[...] = zy[:, O:]

    grid = N_NODES // blk
    return pl.pallas_call(
        body,
        grid=(grid,),
        in_specs=[
            pl.BlockSpec((blk, F), lambda i: (i, 0)),
            pl.BlockSpec((F, 2 * O), lambda i: (0, 0)),
            pl.BlockSpec((1, 2 * O), lambda i: (0, 0)),
        ],
        out_specs=[
            pl.BlockSpec((blk, O), lambda i: (i, 0)),
            pl.BlockSpec((blk, O), lambda i: (i, 0)),
        ],
        out_shape=[
            jax.ShapeDtypeStruct((N_NODES, O), jnp.float32),
            jax.ShapeDtypeStruct((N_NODES, O), jnp.float32),
        ],
    )(X, Wc, bc)


def _sc_gather_add(Z, Yp, idx_flat, seg):
    mesh = plsc.VectorSubcoreMesh(core_axis_name="c", subcore_axis_name="s")

    @functools.partial(
        pl.kernel,
        out_type=jax.ShapeDtypeStruct((N_NODES, O), jnp.float32),
        mesh=mesh,
        scratch_types=[
            pltpu.VMEM((IDX_PER_SUB,), jnp.int32),
            pltpu.VMEM((IDX_PER_SUB,), jnp.int32),
            pltpu.VMEM((IDX_PER_SUB, O), jnp.float32),
            pltpu.VMEM((SUB, O), jnp.float32),
            pltpu.SemaphoreType.DMA,
        ],
    )
    def k(z_hbm, y_hbm, idx_hbm, seg_hbm, out_hbm, idx_v, seg_v, rows_v,
          acc_v, sem):
        wid = lax.axis_index("s") * NC + lax.axis_index("c")
        base = wid * ROWS_PER_W
        nsub = jnp.where(wid == NW - 1, 10, ROWS_PER_W // SUB)
        pltpu.sync_copy(seg_hbm, seg_v)

        def body(c, carry):
            row0 = base + c * SUB
            pltpu.sync_copy(idx_hbm.at[pl.ds(row0 * NUM_NEIGH, IDX_PER_SUB)],
                            idx_v)
            pltpu.async_copy(y_hbm.at[idx_v], rows_v, sem).wait()
            pltpu.sync_copy(z_hbm.at[pl.ds(row0, SUB)], acc_v)
            pltpu.sync_copy(rows_v, acc_v.at[seg_v], add=True)
            pltpu.sync_copy(acc_v, out_hbm.at[pl.ds(row0, SUB)])
            return carry

        lax.fori_loop(0, nsub, body, 0)

    return k(Z, Yp, idx_flat, seg)


def kernel(X, A, neigh_idx, weight, bias):
    del A
    W1 = weight[:F]
    W2 = weight[F:] * (1.0 / NUM_NEIGH)
    Wc = jnp.concatenate([W1, W2], axis=1)
    bc = jnp.concatenate([bias, jnp.zeros((O,), jnp.float32)]).reshape(1, 2 * O)
    Z, Yp = _tc_matmul(X, Wc, bc)
    idx_flat = neigh_idx.astype(jnp.int32).reshape(-1)
    seg = jnp.asarray(_SEG)
    return _sc_gather_add(Z, Yp, idx_flat, seg)


# speedup vs baseline: 2.1209x; 2.1209x over previous
import functools

import jax
import jax.numpy as jnp
import numpy as np
from jax import lax
from jax.experimental import pallas as pl
from jax.experimental.pallas import tpu as pltpu
from jax.experimental.pallas import tpu_sc as plsc

N_NODES = 10000
F = 128
O = 128
NUM_NEIGH = 10

NC = 2
NS = 16
NW = NC * NS

SUB = 8
IDX_PER_SUB = SUB * NUM_NEIGH
ROWS_PER_W = 320

_SEG = (np.repeat(np.arange(SUB, dtype=np.int32), NUM_NEIGH)[None, :]
        + SUB * np.arange(NS, dtype=np.int32)[:, None]).reshape(-1)


def _tc_matmul(X, Wc, bc):
    blk = 2000

    def body(x_ref, w_ref, b_ref, z_ref, y_ref):
        zy = jnp.dot(x_ref[...], w_ref[...],
                     preferred_element_type=jnp.float32) + b_ref[...]
        z_ref[...] = zy[:, :O]
        y_ref[...] = zy[:, O:]

    grid = N_NODES // blk
    return pl.pallas_call(
        body,
        grid=(grid,),
        in_specs=[
            pl.BlockSpec((blk, F), lambda i: (i, 0)),
            pl.BlockSpec((F, 2 * O), lambda i: (0, 0)),
            pl.BlockSpec((1, 2 * O), lambda i: (0, 0)),
        ],
        out_specs=[
            pl.BlockSpec((blk, O), lambda i: (i, 0)),
            pl.BlockSpec((blk, O), lambda i: (i, 0)),
        ],
        out_shape=[
            jax.ShapeDtypeStruct((N_NODES, O), jnp.float32),
            jax.ShapeDtypeStruct((N_NODES, O), jnp.float32),
        ],
    )(X, Wc, bc)


def _sc_gather_add(Z, Yp, idx_flat, seg):
    mesh = plsc.VectorSubcoreMesh(core_axis_name="c", subcore_axis_name="s")

    @functools.partial(
        pl.kernel,
        out_type=jax.ShapeDtypeStruct((N_NODES, O), jnp.float32),
        mesh=mesh,
        scratch_types=[
            pltpu.VMEM((IDX_PER_SUB,), jnp.int32),
            pltpu.VMEM((IDX_PER_SUB,), jnp.int32),
            pltpu.VMEM((IDX_PER_SUB, O), jnp.float32),
            pltpu.VMEM_SHARED((NS * SUB, O), jnp.float32),
            pltpu.SemaphoreType.DMA,
        ],
    )
    def k(z_hbm, y_hbm, idx_hbm, seg_hbm, out_hbm, idx_v, seg_v, rows_v,
          acc_s, sem):
        sid = lax.axis_index("s")
        wid = sid * NC + lax.axis_index("c")
        base = wid * ROWS_PER_W
        nsub = jnp.where(wid == NW - 1, 10, ROWS_PER_W // SUB)
        pltpu.sync_copy(seg_hbm.at[pl.ds(sid * IDX_PER_SUB, IDX_PER_SUB)],
                        seg_v)
        acc0 = sid * SUB

        def body(c, carry):
            row0 = base + c * SUB
            pltpu.sync_copy(idx_hbm.at[pl.ds(row0 * NUM_NEIGH, IDX_PER_SUB)],
                            idx_v)
            pltpu.async_copy(y_hbm.at[idx_v], rows_v, sem).wait()
            pltpu.sync_copy(z_hbm.at[pl.ds(row0, SUB)],
                            acc_s.at[pl.ds(acc0, SUB)])
            pltpu.sync_copy(rows_v, acc_s.at[seg_v], add=True)
            pltpu.sync_copy(acc_s.at[pl.ds(acc0, SUB)],
                            out_hbm.at[pl.ds(row0, SUB)])
            return carry

        lax.fori_loop(0, nsub, body, 0)

    return k(Z, Yp, idx_flat, seg)


def kernel(X, A, neigh_idx, weight, bias):
    del A
    W1 = weight[:F]
    W2 = weight[F:] * (1.0 / NUM_NEIGH)
    Wc = jnp.concatenate([W1, W2], axis=1)
    bc = jnp.concatenate([bias, jnp.zeros((O,), jnp.float32)]).reshape(1, 2 * O)
    Z, Yp = _tc_matmul(X, Wc, bc)
    idx_flat = neigh_idx.astype(jnp.int32).reshape(-1)
    seg = jnp.asarray(_SEG)
    return _sc_gather_add(Z, Yp, idx_flat, seg)
